# baseline (speedup 1.0000x reference)
import functools

import jax
import jax.numpy as jnp
from jax import lax
from jax.experimental import pallas as pl
from jax.experimental.pallas import tpu as pltpu
from jax.experimental.pallas import tpu_sc as plsc

USER = 5000
ITEM = 5000
N = USER + ITEM
E = 320000
D = 128
NUM_LAYERS = 3

NC = 2
NS = 16
NW = NC * NS
EPW = E // NW
CH = 128
NCHUNK = 81
EPWP = NCHUNK * CH
NSLOT = 3
GA = 1
NOUT = NCHUNK // NSLOT
NPAD = N + 16
RPT = 624
TAIL = N - NS * RPT

_mesh = plsc.VectorSubcoreMesh(core_axis_name="c", subcore_axis_name="s")


def _spmm_body(cur_hbm, pk_hbm, val_hbm, out_hbm, *scr):
    rows = scr[0:NSLOT]
    pkv = scr[NSLOT:2 * NSLOT]
    ridx = scr[2 * NSLOT:3 * NSLOT]
    valb = scr[3 * NSLOT:4 * NSLOT]
    acc_sh = scr[4 * NSLOT]
    gsem = scr[4 * NSLOT + 1:5 * NSLOT + 1]
    ssem = scr[5 * NSLOT + 1:6 * NSLOT + 1]
    pksem = scr[6 * NSLOT + 1:7 * NSLOT + 1]
    vsem = scr[7 * NSLOT + 1:8 * NSLOT + 1]

    c = lax.axis_index("c")
    s = lax.axis_index("s")
    wid = s * NC + c

    z16 = jnp.zeros((16,), jnp.float32)

    def zero_row(r, _):
        for j in range(D // 16):
            rows[0][r, pl.ds(j * 16, 16)] = z16
        return 0

    lax.fori_loop(0, CH, zero_row, 0)
    for k in range(RPT // CH):
        pltpu.sync_copy(rows[0], acc_sh.at[pl.ds(s * RPT + k * CH, CH)])
    rem = RPT - (RPT // CH) * CH
    if rem:
        pltpu.sync_copy(rows[0].at[pl.ds(0, rem), :],
                        acc_sh.at[pl.ds(s * RPT + (RPT // CH) * CH, rem)])

    @pl.when(s == NS - 1)
    def _zero_tail():
        pltpu.sync_copy(rows[0].at[pl.ds(0, TAIL), :],
                        acc_sh.at[pl.ds(NS * RPT, TAIL)])

    plsc.subcore_barrier()

    def start_pk(i, b):
        pltpu.async_copy(pk_hbm.at[wid, i], pkv[b], pksem[b])

    def wait_pk(b):
        pltpu.make_async_copy(pk_hbm.at[wid, 0], pkv[b], pksem[b]).wait()

    def start_val(i, b):
        pltpu.async_copy(val_hbm.at[wid, i], valb[b], vsem[b])

    def wait_val(b):
        pltpu.make_async_copy(val_hbm.at[wid, 0], valb[b], vsem[b]).wait()

    def start_gather(b):
        pltpu.async_copy(cur_hbm.at[pkv[b].at[0]], rows[b], gsem[b])

    def wait_gather(b):
        pltpu.make_async_copy(cur_hbm.at[pkv[b].at[0]], rows[b],
                              gsem[b]).wait()

    def start_scatter(b):
        pltpu.async_copy(rows[b], acc_sh.at[ridx[b]], ssem[b], add=True)

    def wait_scatter(b):
        pltpu.make_async_copy(rows[b], acc_sh.at[ridx[b]], ssem[b]).wait()

    def copy_ridx(b):
        for g in range(CH // 16):
            sl = pl.ds(g * 16, 16)
            ridx[b][sl] = pkv[b][1, sl]

    def mul(b):
        def mul_body(g, _):
            vv = valb[b][pl.ds(g * 16, 16)]
            for k in range(16):
                splat = jnp.full((16,), vv[k], jnp.float32)
                e = g * 16 + k
                for j in range(D // 16):
                    sl = pl.ds(j * 16, 16)
                    rows[b][e, sl] = rows[b][e, sl] * splat
            return 0

        lax.fori_loop(0, CH // 16, mul_body, 0)

    pltpu.sync_copy(pk_hbm.at[wid, 0], pkv[0])
    for j in range(1, NSLOT):
        start_pk(j, j)
    for j in range(NSLOT):
        start_val(j, j)
    start_gather(0)
    for j in range(1, GA):
        wait_pk(j)
        start_gather(j)

    def outer_body(t, _):
        for b in range(NSLOT):
            i = NSLOT * t + b
            sb = (b + GA) % NSLOT

            wait_gather(b)
            copy_ridx(b)

            def launch_ahead():
                wait_pk(sb)
                wait_scatter(sb)
                start_gather(sb)

            if b < NSLOT - GA:
                @pl.when(t >= 1)
                def _la():
                    launch_ahead()

                @pl.when(t == 0)
                def _la0():
                    wait_pk(sb)
                    start_gather(sb)
            else:
                @pl.when(t < NOUT - 1)
                def _la2():
                    launch_ahead()

            wait_val(b)
            mul(b)
            start_scatter(b)

            @pl.when(t < NOUT - 1)
            def _pf():
                start_pk(i + NSLOT, b)
                start_val(i + NSLOT, b)
        return 0

    lax.fori_loop(0, NOUT, outer_body, 0)

    for b in range(NSLOT):
        wait_scatter(b)
    plsc.subcore_barrier()

    pltpu.sync_copy(acc_sh.at[pl.ds(s * RPT, RPT)],
                    out_hbm.at[c, pl.ds(s * RPT, RPT)])

    @pl.when(s == NS - 1)
    def _write_tail():
        pltpu.sync_copy(acc_sh.at[pl.ds(NS * RPT, TAIL)],
                        out_hbm.at[c, pl.ds(NS * RPT, TAIL)])


_spmm = pl.kernel(
    _spmm_body,
    out_type=jax.ShapeDtypeStruct((NC, N, D), jnp.float32),
    mesh=_mesh,
    scratch_types=(
        [pltpu.VMEM((CH, D), jnp.float32) for _ in range(NSLOT)]
        + [pltpu.VMEM((2, CH), jnp.int32) for _ in range(NSLOT)]
        + [pltpu.VMEM((CH,), jnp.int32) for _ in range(NSLOT)]
        + [pltpu.VMEM((CH,), jnp.float32) for _ in range(NSLOT)]
        + [pltpu.VMEM_SHARED((NPAD, D), jnp.float32)]
        + [pltpu.SemaphoreType.DMA for _ in range(4 * NSLOT)]
    ),
)


def _combine_body(p_ref, t_ref, cur_ref, tot_ref):
    layer = p_ref[0] + p_ref[1]
    cur_ref[...] = layer
    tot_ref[...] = t_ref[...] + layer


_BR = 2000


def _combine(part, tot):
    grid = (N // _BR,)
    return pl.pallas_call(
        _combine_body,
        grid=grid,
        in_specs=[
            pl.BlockSpec((NC, _BR, D), lambda i: (0, i, 0)),
            pl.BlockSpec((_BR, D), lambda i: (i, 0)),
        ],
        out_specs=[
            pl.BlockSpec((_BR, D), lambda i: (i, 0)),
            pl.BlockSpec((_BR, D), lambda i: (i, 0)),
        ],
        out_shape=[
            jax.ShapeDtypeStruct((N, D), jnp.float32),
            jax.ShapeDtypeStruct((N, D), jnp.float32),
        ],
    )(part, tot)


def kernel(adj_values, uEmbeds, iEmbeds, edge_index):
    embeds = jnp.concatenate([uEmbeds, iEmbeds], axis=0)
    npadw = EPWP - EPW
    row = jnp.concatenate(
        [edge_index[0].astype(jnp.int32).reshape(NW, EPW),
         jnp.full((NW, npadw), N, jnp.int32)], axis=1)
    col = jnp.concatenate(
        [edge_index[1].astype(jnp.int32).reshape(NW, EPW),
         jnp.zeros((NW, npadw), jnp.int32)], axis=1)
    pk = jnp.concatenate(
        [col.reshape(NW, NCHUNK, 1, CH), row.reshape(NW, NCHUNK, 1, CH)],
        axis=2)
    vals = jnp.concatenate(
        [adj_values.astype(jnp.float32).reshape(NW, EPW),
         jnp.zeros((NW, npadw), jnp.float32)], axis=1).reshape(
             NW, NCHUNK, CH)

    cur = embeds
    tot = embeds
    for _ in range(NUM_LAYERS):
        part = _spmm(cur, pk, vals)
        cur, tot = _combine(part, tot)
    return tot[:USER], tot[USER:]


# speedup vs baseline: 2.3515x; 2.3515x over previous
import jax
import jax.numpy as jnp
from jax import lax
from jax.experimental import pallas as pl
from jax.experimental.pallas import tpu as pltpu
from jax.experimental.pallas import tpu_sc as plsc

USER = 5000
ITEM = 5000
N = USER + ITEM
E = 320000
D = 128
NUM_LAYERS = 3

NC = 2
NS = 16
NW = NC * NS
EPW = E // NW
CH = 112
NCHUNK = 90
EPWP = NCHUNK * CH
NSLOT = 3
GA = 1
NOUT = NCHUNK // NSLOT
NPAD = N + 16
RPT = 624
TAIL = N - NS * RPT

_mesh = plsc.VectorSubcoreMesh(core_axis_name="c", subcore_axis_name="s")


def _spmm_body(cur_hbm, pk_hbm, val_hbm, out_hbm, *scr):
    rows = scr[0:NSLOT]
    pkv = scr[NSLOT:2 * NSLOT]
    ridx = scr[2 * NSLOT:3 * NSLOT]
    valb = scr[3 * NSLOT:4 * NSLOT]
    acc_sh = scr[4 * NSLOT]
    gsem = scr[4 * NSLOT + 1:5 * NSLOT + 1]
    ssem = scr[5 * NSLOT + 1:6 * NSLOT + 1]
    pksem = scr[6 * NSLOT + 1:7 * NSLOT + 1]
    vsem = scr[7 * NSLOT + 1:8 * NSLOT + 1]

    c = lax.axis_index("c")
    s = lax.axis_index("s")
    wid = s * NC + c

    z16 = jnp.zeros((16,), jnp.float32)

    def zero_row(r, _):
        for j in range(D // 16):
            rows[0][r, pl.ds(j * 16, 16)] = z16
        return 0

    lax.fori_loop(0, CH, zero_row, 0)
    for k in range(RPT // CH):
        pltpu.sync_copy(rows[0], acc_sh.at[pl.ds(s * RPT + k * CH, CH)])
    rem = RPT - (RPT // CH) * CH
    if rem:
        pltpu.sync_copy(rows[0].at[pl.ds(0, rem), :],
                        acc_sh.at[pl.ds(s * RPT + (RPT // CH) * CH, rem)])

    @pl.when(s == NS - 1)
    def _zero_tail():
        pltpu.sync_copy(rows[0].at[pl.ds(0, TAIL), :],
                        acc_sh.at[pl.ds(NS * RPT, TAIL)])

    plsc.subcore_barrier()

    def start_pk(i, b):
        pltpu.async_copy(pk_hbm.at[wid, i], pkv[b], pksem[b])

    def wait_pk(b):
        pltpu.make_async_copy(pk_hbm.at[wid, 0], pkv[b], pksem[b]).wait()

    def start_val(i, b):
        pltpu.async_copy(val_hbm.at[wid, i], valb[b], vsem[b])

    def wait_val(b):
        pltpu.make_async_copy(val_hbm.at[wid, 0], valb[b], vsem[b]).wait()

    def start_gather(b):
        pltpu.async_copy(cur_hbm.at[pkv[b].at[0]], rows[b], gsem[b])

    def wait_gather(b):
        pltpu.make_async_copy(cur_hbm.at[pkv[b].at[0]], rows[b],
                              gsem[b]).wait()

    def start_scatter(b):
        pltpu.async_copy(rows[b], acc_sh.at[ridx[b]], ssem[b], add=True)

    def wait_scatter(b):
        pltpu.make_async_copy(rows[b], acc_sh.at[ridx[b]], ssem[b]).wait()

    def copy_ridx(b):
        for g in range(CH // 16):
            sl = pl.ds(g * 16, 16)
            ridx[b][sl] = pkv[b][1, sl]

    def mul(b):
        def mul_body(g, _):
            vv = valb[b][pl.ds(g * 16, 16)]
            for k in range(16):
                splat = jnp.full((16,), vv[k], jnp.float32)
                e = g * 16 + k
                for j in range(D // 16):
                    sl = pl.ds(j * 16, 16)
                    rows[b][e, sl] = rows[b][e, sl] * splat
            return 0

        lax.fori_loop(0, CH // 16, mul_body, 0)

    pltpu.sync_copy(pk_hbm.at[wid, 0], pkv[0])
    for j in range(1, NSLOT):
        start_pk(j, j)
    for j in range(NSLOT):
        start_val(j, j)
    start_gather(0)
    for j in range(1, GA):
        wait_pk(j)
        start_gather(j)

    def outer_body(t, _):
        for b in range(NSLOT):
            i = NSLOT * t + b
            sb = (b + GA) % NSLOT

            wait_gather(b)
            copy_ridx(b)

            def launch_ahead():
                wait_pk(sb)
                wait_scatter(sb)
                start_gather(sb)

            if b < NSLOT - GA:
                @pl.when(t >= 1)
                def _la():
                    launch_ahead()

                @pl.when(t == 0)
                def _la0():
                    wait_pk(sb)
                    start_gather(sb)
            else:
                @pl.when(t < NOUT - 1)
                def _la2():
                    launch_ahead()

            wait_val(b)
            mul(b)
            start_scatter(b)

            @pl.when(t < NOUT - 1)
            def _pf():
                start_pk(i + NSLOT, b)
                start_val(i + NSLOT, b)
        return 0

    lax.fori_loop(0, NOUT, outer_body, 0)

    for b in range(NSLOT):
        wait_scatter(b)
    plsc.subcore_barrier()

    pltpu.sync_copy(acc_sh.at[pl.ds(s * RPT, RPT)],
                    out_hbm.at[c, pl.ds(s * RPT, RPT)])

    @pl.when(s == NS - 1)
    def _write_tail():
        pltpu.sync_copy(acc_sh.at[pl.ds(NS * RPT, TAIL)],
                        out_hbm.at[c, pl.ds(NS * RPT, TAIL)])


_spmm = pl.kernel(
    _spmm_body,
    out_type=jax.ShapeDtypeStruct((NC, N, D), jnp.float32),
    mesh=_mesh,
    scratch_types=(
        [pltpu.VMEM((CH, D), jnp.float32) for _ in range(NSLOT)]
        + [pltpu.VMEM((2, CH), jnp.int32) for _ in range(NSLOT)]
        + [pltpu.VMEM((CH,), jnp.int32) for _ in range(NSLOT)]
        + [pltpu.VMEM((CH,), jnp.float32) for _ in range(NSLOT)]
        + [pltpu.VMEM_SHARED((NPAD, D), jnp.float32)]
        + [pltpu.SemaphoreType.DMA for _ in range(4 * NSLOT)]
    ),
)


def _combine_body(p_ref, t_ref, cur_ref, tot_ref):
    layer = p_ref[0] + p_ref[1]
    cur_ref[...] = layer
    tot_ref[...] = t_ref[...] + layer


_BR = 2000


def _combine(part, tot):
    grid = (N // _BR,)
    return pl.pallas_call(
        _combine_body,
        grid=grid,
        in_specs=[
            pl.BlockSpec((NC, _BR, D), lambda i: (0, i, 0)),
            pl.BlockSpec((_BR, D), lambda i: (i, 0)),
        ],
        out_specs=[
            pl.BlockSpec((_BR, D), lambda i: (i, 0)),
            pl.BlockSpec((_BR, D), lambda i: (i, 0)),
        ],
        out_shape=[
            jax.ShapeDtypeStruct((N, D), jnp.float32),
            jax.ShapeDtypeStruct((N, D), jnp.float32),
        ],
    )(part, tot)


def kernel(adj_values, uEmbeds, iEmbeds, edge_index):
    embeds = jnp.concatenate([uEmbeds, iEmbeds], axis=0)
    npadw = EPWP - EPW
    row = jnp.concatenate(
        [edge_index[0].astype(jnp.int32).reshape(NW, EPW),
         jnp.full((NW, npadw), N, jnp.int32)], axis=1)
    col = jnp.concatenate(
        [edge_index[1].astype(jnp.int32).reshape(NW, EPW),
         jnp.zeros((NW, npadw), jnp.int32)], axis=1)
    pk = jnp.concatenate(
        [col.reshape(NW, NCHUNK, 1, CH), row.reshape(NW, NCHUNK, 1, CH)],
        axis=2)
    vals = jnp.concatenate(
        [adj_values.astype(jnp.float32).reshape(NW, EPW),
         jnp.zeros((NW, npadw), jnp.float32)], axis=1).reshape(
             NW, NCHUNK, CH)

    cur = embeds
    tot = embeds
    for _ in range(NUM_LAYERS):
        part = _spmm(cur, pk, vals)
        cur, tot = _combine(part, tot)
    return tot[:USER], tot[USER:]


# speedup vs baseline: 2.3559x; 1.0019x over previous
import jax
import jax.numpy as jnp
from jax import lax
from jax.experimental import pallas as pl
from jax.experimental.pallas import tpu as pltpu
from jax.experimental.pallas import tpu_sc as plsc

USER = 5000
ITEM = 5000
N = USER + ITEM
E = 320000
D = 128
NUM_LAYERS = 3

NC = 2
NS = 16
NW = NC * NS
EPW = E // NW
CH = 112
NCHUNK = 90
EPWP = NCHUNK * CH
NSLOT = 3
GA = 1
NOUT = NCHUNK // NSLOT
NPAD = N + 16
RPT = 624
TAIL = N - NS * RPT

_mesh = plsc.VectorSubcoreMesh(core_axis_name="c", subcore_axis_name="s")


def _spmm_body(cur_hbm, pk_hbm, val_hbm, out_hbm, *scr):
    rows = scr[0:NSLOT]
    pkv = scr[NSLOT:2 * NSLOT]
    ridx = scr[2 * NSLOT:3 * NSLOT]
    valb = scr[3 * NSLOT:4 * NSLOT]
    acc_sh = scr[4 * NSLOT]
    gsem = scr[4 * NSLOT + 1:5 * NSLOT + 1]
    ssem = scr[5 * NSLOT + 1:6 * NSLOT + 1]
    pksem = scr[6 * NSLOT + 1:7 * NSLOT + 1]
    vsem = scr[7 * NSLOT + 1:8 * NSLOT + 1]

    c = lax.axis_index("c")
    s = lax.axis_index("s")
    wid = s * NC + c

    z16 = jnp.zeros((16,), jnp.float32)

    def zero_row(r, _):
        for j in range(D // 16):
            rows[0][r, pl.ds(j * 16, 16)] = z16
        return 0

    lax.fori_loop(0, CH, zero_row, 0)
    for k in range(RPT // CH):
        pltpu.sync_copy(rows[0], acc_sh.at[pl.ds(s * RPT + k * CH, CH)])
    rem = RPT - (RPT // CH) * CH
    if rem:
        pltpu.sync_copy(rows[0].at[pl.ds(0, rem), :],
                        acc_sh.at[pl.ds(s * RPT + (RPT // CH) * CH, rem)])

    @pl.when(s == NS - 1)
    def _zero_tail():
        pltpu.sync_copy(rows[0].at[pl.ds(0, TAIL), :],
                        acc_sh.at[pl.ds(NS * RPT, TAIL)])

    plsc.subcore_barrier()

    def start_pk(i, b):
        pltpu.async_copy(pk_hbm.at[wid, i], pkv[b], pksem[b])

    def wait_pk(b):
        pltpu.make_async_copy(pk_hbm.at[wid, 0], pkv[b], pksem[b]).wait()

    def start_val(i, b):
        pltpu.async_copy(val_hbm.at[wid, i], valb[b], vsem[b])

    def wait_val(b):
        pltpu.make_async_copy(val_hbm.at[wid, 0], valb[b], vsem[b]).wait()

    def start_gather(b):
        pltpu.async_copy(cur_hbm.at[pkv[b].at[0]], rows[b], gsem[b])

    def wait_gather(b):
        pltpu.make_async_copy(cur_hbm.at[pkv[b].at[0]], rows[b],
                              gsem[b]).wait()

    def start_scatter(b):
        pltpu.async_copy(rows[b], acc_sh.at[ridx[b]], ssem[b], add=True)

    def wait_scatter(b):
        pltpu.make_async_copy(rows[b], acc_sh.at[ridx[b]], ssem[b]).wait()

    def copy_ridx(b):
        for g in range(CH // 16):
            sl = pl.ds(g * 16, 16)
            ridx[b][sl] = pkv[b][1, sl]

    def mul(b):
        def mul_body(g, _):
            vv = valb[b][pl.ds(g * 16, 16)]
            for k in range(16):
                splat = jnp.full((16,), vv[k], jnp.float32)
                e = g * 16 + k
                for j in range(D // 16):
                    sl = pl.ds(j * 16, 16)
                    rows[b][e, sl] = rows[b][e, sl] * splat
            return 0

        lax.fori_loop(0, CH // 16, mul_body, 0)

    pltpu.sync_copy(pk_hbm.at[wid, 0], pkv[0])
    for j in range(1, NSLOT):
        start_pk(j, j)
    for j in range(NSLOT):
        start_val(j, j)
    start_gather(0)
    for j in range(1, GA):
        wait_pk(j)
        start_gather(j)

    def outer_body(t, _):
        for b in range(NSLOT):
            i = NSLOT * t + b
            sb = (b + GA) % NSLOT

            wait_gather(b)

            def launch_ahead():
                wait_pk(sb)
                wait_scatter(sb)
                start_gather(sb)

            if b < NSLOT - GA:
                @pl.when(t >= 1)
                def _la():
                    launch_ahead()

                @pl.when(t == 0)
                def _la0():
                    wait_pk(sb)
                    start_gather(sb)
            else:
                @pl.when(t < NOUT - 1)
                def _la2():
                    launch_ahead()

            copy_ridx(b)
            wait_val(b)
            mul(b)
            start_scatter(b)

            @pl.when(t < NOUT - 1)
            def _pf():
                start_pk(i + NSLOT, b)
                start_val(i + NSLOT, b)
        return 0

    lax.fori_loop(0, NOUT, outer_body, 0)

    for b in range(NSLOT):
        wait_scatter(b)
    plsc.subcore_barrier()

    pltpu.sync_copy(acc_sh.at[pl.ds(s * RPT, RPT)],
                    out_hbm.at[c, pl.ds(s * RPT, RPT)])

    @pl.when(s == NS - 1)
    def _write_tail():
        pltpu.sync_copy(acc_sh.at[pl.ds(NS * RPT, TAIL)],
                        out_hbm.at[c, pl.ds(NS * RPT, TAIL)])


_spmm = pl.kernel(
    _spmm_body,
    out_type=jax.ShapeDtypeStruct((NC, N, D), jnp.float32),
    mesh=_mesh,
    scratch_types=(
        [pltpu.VMEM((CH, D), jnp.float32) for _ in range(NSLOT)]
        + [pltpu.VMEM((2, CH), jnp.int32) for _ in range(NSLOT)]
        + [pltpu.VMEM((CH,), jnp.int32) for _ in range(NSLOT)]
        + [pltpu.VMEM((CH,), jnp.float32) for _ in range(NSLOT)]
        + [pltpu.VMEM_SHARED((NPAD, D), jnp.float32)]
        + [pltpu.SemaphoreType.DMA for _ in range(4 * NSLOT)]
    ),
)


def _combine_body(p_ref, t_ref, cur_ref, tot_ref):
    layer = p_ref[0] + p_ref[1]
    cur_ref[...] = layer
    tot_ref[...] = t_ref[...] + layer


_BR = 2000


def _combine(part, tot):
    grid = (N // _BR,)
    return pl.pallas_call(
        _combine_body,
        grid=grid,
        in_specs=[
            pl.BlockSpec((NC, _BR, D), lambda i: (0, i, 0)),
            pl.BlockSpec((_BR, D), lambda i: (i, 0)),
        ],
        out_specs=[
            pl.BlockSpec((_BR, D), lambda i: (i, 0)),
            pl.BlockSpec((_BR, D), lambda i: (i, 0)),
        ],
        out_shape=[
            jax.ShapeDtypeStruct((N, D), jnp.float32),
            jax.ShapeDtypeStruct((N, D), jnp.float32),
        ],
    )(part, tot)


def kernel(adj_values, uEmbeds, iEmbeds, edge_index):
    embeds = jnp.concatenate([uEmbeds, iEmbeds], axis=0)
    npadw = EPWP - EPW
    row = jnp.concatenate(
        [edge_index[0].astype(jnp.int32).reshape(NW, EPW),
         jnp.full((NW, npadw), N, jnp.int32)], axis=1)
    col = jnp.concatenate(
        [edge_index[1].astype(jnp.int32).reshape(NW, EPW),
         jnp.zeros((NW, npadw), jnp.int32)], axis=1)
    pk = jnp.concatenate(
        [col.reshape(NW, NCHUNK, 1, CH), row.reshape(NW, NCHUNK, 1, CH)],
        axis=2)
    vals = jnp.concatenate(
        [adj_values.astype(jnp.float32).reshape(NW, EPW),
         jnp.zeros((NW, npadw), jnp.float32)], axis=1).reshape(
             NW, NCHUNK, CH)

    cur = embeds
    tot = embeds
    for _ in range(NUM_LAYERS):
        part = _spmm(cur, pk, vals)
        cur, tot = _combine(part, tot)
    return tot[:USER], tot[USER:]
